# Initial kernel scaffold; baseline (speedup 1.0000x reference)
import jax
import jax.numpy as jnp
from jax.experimental import pallas as pl


def kernel(x, params, edge_index):
    raise NotImplementedError("write your pallas kernel here")


# speedup vs baseline: 3.8127x; 3.8127x over previous
import functools

import jax
import jax.numpy as jnp
from jax import lax
from jax.experimental import pallas as pl
from jax.experimental.pallas import tpu as pltpu
from jax.experimental.pallas import tpu_sc as plsc

N = 10000
E = 160000
D = 256
HD = 128
NS = 16
CHUNK = 128
C = -(-(E // NS) // CHUNK)
EPS = NS * C * CHUNK
ROWS_PER_SUB = 632
NP = NS * ROWS_PER_SUB
GARBAGE = N


def _sc_agg_body(h_hbm, src_hbm, dst_hbm, out_hbm, sidx, didx, rowbuf, shared, gsem):
    cid = lax.axis_index("c")
    sid = lax.axis_index("s")
    pltpu.sync_copy(src_hbm.at[cid, sid], sidx)
    pltpu.sync_copy(dst_hbm.at[sid], didx)
    base = sid * ROWS_PER_SUB
    hbase = cid * NP + base
    pltpu.sync_copy(h_hbm.at[pl.ds(hbase, ROWS_PER_SUB)],
                    shared.at[pl.ds(base, ROWS_PER_SUB)])
    plsc.subcore_barrier()

    def chunk(j, carry):
        pltpu.async_copy(h_hbm.at[sidx.at[j]], rowbuf, gsem).wait()
        pltpu.sync_copy(rowbuf, shared.at[didx.at[j]], add=True)
        return carry

    lax.fori_loop(0, C, chunk, 0, unroll=False)
    plsc.subcore_barrier()
    pltpu.sync_copy(shared.at[pl.ds(base, ROWS_PER_SUB)],
                    out_hbm.at[pl.ds(hbase, ROWS_PER_SUB)])


_sc_agg = functools.partial(
    pl.kernel,
    out_type=jax.ShapeDtypeStruct((2 * NP, HD), jnp.float32),
    mesh=plsc.VectorSubcoreMesh(core_axis_name="c", subcore_axis_name="s"),
    scratch_types=[
        pltpu.VMEM((C, CHUNK), jnp.int32),
        pltpu.VMEM((C, CHUNK), jnp.int32),
        pltpu.VMEM((CHUNK, HD), jnp.float32),
        pltpu.VMEM_SHARED((NP, HD), jnp.float32),
        pltpu.SemaphoreType.DMA,
    ],
)(_sc_agg_body)


def _bn_relu(h, g, b):
    mu = jnp.mean(h, axis=0, keepdims=True)
    var = jnp.mean((h - mu) * (h - mu), axis=0, keepdims=True)
    h = (h - mu) * lax.rsqrt(var + 1e-5) * g + b
    return jnp.maximum(h, 0.0)


def _tc_mlp_body(s_ref, wa_ref, ba_ref, ga_ref, bea_ref, wb_ref, bb_ref,
                 gb_ref, beb_ref, out_ref):
    s = jnp.concatenate([s_ref[:N], s_ref[NP:NP + N]], axis=1)
    h = jnp.dot(s, wa_ref[...], preferred_element_type=jnp.float32) + ba_ref[...]
    h = _bn_relu(h, ga_ref[...], bea_ref[...])
    h = jnp.dot(h, wb_ref[...], preferred_element_type=jnp.float32) + bb_ref[...]
    h = _bn_relu(h, gb_ref[...], beb_ref[...])
    out_ref[:N] = h[:, :HD]
    out_ref[NP:NP + N] = h[:, HD:]


_tc_mlp = pl.pallas_call(
    _tc_mlp_body,
    out_shape=jax.ShapeDtypeStruct((2 * NP, HD), jnp.float32),
)


def _tc_final_body(s_ref, w_ref, b_ref, g_ref, be_ref, out_ref):
    s = jnp.concatenate([s_ref[:N], s_ref[NP:NP + N]], axis=1)
    h = jnp.dot(s, w_ref[...], preferred_element_type=jnp.float32) + b_ref[...]
    out_ref[...] = _bn_relu(h, g_ref[...], be_ref[...])


_tc_final = pl.pallas_call(
    _tc_final_body,
    out_shape=jax.ShapeDtypeStruct((N, D), jnp.float32),
)


def kernel(x, params, edge_index):
    src = edge_index[0]
    dst = edge_index[1]
    pad = EPS - E
    src_p = jnp.concatenate([src, jnp.zeros((pad,), jnp.int32)])
    dst_p = jnp.concatenate([dst, jnp.full((pad,), GARBAGE, jnp.int32)])
    src3 = (src_p[None, :] + (jnp.arange(2, dtype=jnp.int32) * NP)[:, None]
            ).reshape(2, NS, C, CHUNK)
    dst2 = dst_p.reshape(NS, C, CHUNK)

    padrows = jnp.zeros((NP - N, HD), jnp.float32)
    h = jnp.concatenate([x[:, :HD], padrows, x[:, HD:], padrows], axis=0)

    def p2(name):
        return params[name].reshape(1, -1)

    for i in range(2):
        s = _sc_agg(h, src3, dst2)
        h = _tc_mlp(s, params['W%da' % i], p2('b%da' % i), p2('g%da' % i),
                    p2('be%da' % i), params['W%db' % i], p2('b%db' % i),
                    p2('g%db' % i), p2('be%db' % i))
    s = _sc_agg(h, src3, dst2)
    return _tc_final(s, params['W2'], p2('b2'), p2('g2'), p2('be2'))
